# Initial kernel scaffold; baseline (speedup 1.0000x reference)
import jax
import jax.numpy as jnp
from jax.experimental import pallas as pl


def kernel(features, labels, epoch, running_mean_last_epoch, running_var_last_epoch, smoothed_mean_last_epoch, smoothed_var_last_epoch):
    raise NotImplementedError("write your pallas kernel here")


# speedup vs baseline: 7.7668x; 7.7668x over previous
import functools

import jax
import jax.numpy as jnp
from jax import lax
from jax.experimental import pallas as pl
from jax.experimental.pallas import tpu as pltpu

BUCKETS = 50
D = 2048
BLOCK_N = 512


def _fds_kernel(labels_ref, features_ref, m1_ref, v1_ref, m2_ref, v2_ref,
                out_ref, scale_ref, bias_ref):
    @pl.when(pl.program_id(0) == 0)
    def _prep():
        scale = jnp.sqrt(jnp.clip(v2_ref[...] / v1_ref[...], 0.5, 2.0))
        scale_ref[...] = scale
        bias_ref[...] = m2_ref[...] - m1_ref[...] * scale

    labels = labels_ref[0, 0, :]
    idx = jnp.where(labels <= 1.0, BUCKETS - 1, 0).astype(jnp.int32)

    onehot = (idx[:, None] == lax.broadcasted_iota(jnp.int32, (1, BUCKETS), 1)
              ).astype(jnp.float32)
    row_scale = jnp.dot(onehot, scale_ref[...],
                        preferred_element_type=jnp.float32)
    row_bias = jnp.dot(onehot, bias_ref[...],
                       preferred_element_type=jnp.float32)
    out_ref[...] = features_ref[...] * row_scale + row_bias


@functools.partial(jax.jit, static_argnames=())
def kernel(features, labels, epoch, running_mean_last_epoch,
           running_var_last_epoch, smoothed_mean_last_epoch,
           smoothed_var_last_epoch):
    n = features.shape[0]
    grid = n // BLOCK_N
    smooth = epoch >= 1
    m1 = jnp.where(smooth, running_mean_last_epoch, 0.0)
    v1 = jnp.where(smooth, running_var_last_epoch, 1.0)
    m2 = jnp.where(smooth, smoothed_mean_last_epoch, 0.0)
    v2 = jnp.where(smooth, smoothed_var_last_epoch, 1.0)
    labels3 = labels.reshape(grid, 1, BLOCK_N)

    table_spec = pl.BlockSpec((BUCKETS, D), lambda i: (0, 0))
    return pl.pallas_call(
        _fds_kernel,
        grid=(grid,),
        in_specs=[
            pl.BlockSpec((1, 1, BLOCK_N), lambda i: (i, 0, 0)),
            pl.BlockSpec((BLOCK_N, D), lambda i: (i, 0)),
            table_spec, table_spec, table_spec, table_spec,
        ],
        out_specs=pl.BlockSpec((BLOCK_N, D), lambda i: (i, 0)),
        out_shape=jax.ShapeDtypeStruct((n, D), jnp.float32),
        scratch_shapes=[
            pltpu.VMEM((BUCKETS, D), jnp.float32),
            pltpu.VMEM((BUCKETS, D), jnp.float32),
        ],
    )(labels3, features, m1, v1, m2, v2)


# speedup vs baseline: 8.0189x; 1.0325x over previous
import functools

import jax
import jax.numpy as jnp
from jax import lax
from jax.experimental import pallas as pl
from jax.experimental.pallas import tpu as pltpu

BUCKETS = 50
D = 2048
BLOCK_N = 1024


def _fds_kernel(labels_ref, features_ref, m1_ref, v1_ref, m2_ref, v2_ref,
                out_ref, scale_ref, bias_ref):
    @pl.when(pl.program_id(0) == 0)
    def _prep():
        scale = jnp.sqrt(jnp.clip(v2_ref[...] / v1_ref[...], 0.5, 2.0))
        scale_ref[...] = scale
        bias_ref[...] = m2_ref[...] - m1_ref[...] * scale

    labels = labels_ref[0, 0, :]
    idx = jnp.where(labels <= 1.0, BUCKETS - 1, 0).astype(jnp.int32)

    onehot = (idx[:, None] == lax.broadcasted_iota(jnp.int32, (1, BUCKETS), 1)
              ).astype(jnp.float32)
    row_scale = jnp.dot(onehot, scale_ref[...],
                        preferred_element_type=jnp.float32)
    row_bias = jnp.dot(onehot, bias_ref[...],
                       preferred_element_type=jnp.float32)
    out_ref[...] = features_ref[...] * row_scale + row_bias


@functools.partial(jax.jit, static_argnames=())
def kernel(features, labels, epoch, running_mean_last_epoch,
           running_var_last_epoch, smoothed_mean_last_epoch,
           smoothed_var_last_epoch):
    n = features.shape[0]
    grid = n // BLOCK_N
    smooth = epoch >= 1
    m1 = jnp.where(smooth, running_mean_last_epoch, 0.0)
    v1 = jnp.where(smooth, running_var_last_epoch, 1.0)
    m2 = jnp.where(smooth, smoothed_mean_last_epoch, 0.0)
    v2 = jnp.where(smooth, smoothed_var_last_epoch, 1.0)
    labels3 = labels.reshape(grid, 1, BLOCK_N)

    table_spec = pl.BlockSpec((BUCKETS, D), lambda i: (0, 0))
    return pl.pallas_call(
        _fds_kernel,
        grid=(grid,),
        in_specs=[
            pl.BlockSpec((1, 1, BLOCK_N), lambda i: (i, 0, 0)),
            pl.BlockSpec((BLOCK_N, D), lambda i: (i, 0)),
            table_spec, table_spec, table_spec, table_spec,
        ],
        out_specs=pl.BlockSpec((BLOCK_N, D), lambda i: (i, 0)),
        out_shape=jax.ShapeDtypeStruct((n, D), jnp.float32),
        scratch_shapes=[
            pltpu.VMEM((BUCKETS, D), jnp.float32),
            pltpu.VMEM((BUCKETS, D), jnp.float32),
        ],
    )(labels3, features, m1, v1, m2, v2)
